# Initial kernel scaffold; baseline (speedup 1.0000x reference)
import jax
import jax.numpy as jnp
from jax.experimental import pallas as pl


def kernel(word_idx, table, ngram_idx, ngram_cnt):
    raise NotImplementedError("write your pallas kernel here")


# speedup vs baseline: 9.2862x; 9.2862x over previous
import functools

import jax
import jax.numpy as jnp
from jax import lax
from jax.experimental import pallas as pl
from jax.experimental.pallas import tpu as pltpu
from jax.experimental.pallas import tpu_sc as plsc

_info = plsc.get_sparse_core_info()
_NC, _NS, _L = _info.num_cores, _info.num_subcores, _info.num_lanes
_NW = _NC * _NS

_V = 64
_E = 64
_GPAD = 8
_WPT = _V // _NW
_TOK = 1024 * 20
_TPT = _TOK // _NW
_CHUNK = 128
_NCHUNK = _TPT // _CHUNK

_mesh = plsc.VectorSubcoreMesh(core_axis_name="c", subcore_axis_name="s")
_params = pltpu.CompilerParams(use_tc_tiling_on_sc=False)


@functools.partial(
    pl.kernel,
    mesh=_mesh,
    compiler_params=_params,
    out_type=jax.ShapeDtypeStruct((_V, _E), jnp.float32),
    scratch_types=[
        pltpu.VMEM((_WPT * _GPAD,), jnp.int32),
        pltpu.VMEM((_WPT * _GPAD, _E), jnp.float32),
        pltpu.VMEM((_WPT, _E), jnp.float32),
        pltpu.VMEM((_WPT, _E), jnp.float32),
        pltpu.SemaphoreType.DMA,
    ],
)
def _build_word_emb(table_hbm, idxp_hbm, cntb_hbm, emb_hbm,
                    idx_v, rows_v, cnt_v, emb_v, sem):
    wid = lax.axis_index("s") * _NC + lax.axis_index("c")
    pltpu.sync_copy(idxp_hbm.at[wid], idx_v)
    pltpu.async_copy(table_hbm.at[idx_v], rows_v, sem).wait()
    pltpu.sync_copy(cntb_hbm.at[pl.ds(wid * _WPT, _WPT)], cnt_v)
    for wloc in range(_WPT):
        for c in range(_E // _L):
            acc = rows_v[_GPAD * wloc, pl.ds(c * _L, _L)]
            for l in range(1, _GPAD):
                acc = acc + rows_v[_GPAD * wloc + l, pl.ds(c * _L, _L)]
            emb_v[wloc, pl.ds(c * _L, _L)] = acc / cnt_v[wloc, pl.ds(c * _L, _L)]
    pltpu.sync_copy(emb_v, emb_hbm.at[pl.ds(wid * _WPT, _WPT)])


@functools.partial(
    pl.kernel,
    mesh=_mesh,
    compiler_params=_params,
    out_type=jax.ShapeDtypeStruct((_TOK, _E), jnp.float32),
    scratch_types=[
        pltpu.VMEM((_NCHUNK, _CHUNK), jnp.int32),
        pltpu.VMEM((_TPT, _E), jnp.float32),
        pltpu.SemaphoreType.DMA,
    ],
)
def _expand(emb_hbm, idx_hbm, out_hbm, idx_v, rows_v, sem):
    wid = lax.axis_index("s") * _NC + lax.axis_index("c")
    pltpu.sync_copy(idx_hbm.at[wid], idx_v)
    copies = []
    for j in range(_NCHUNK):
        copies.append(
            pltpu.async_copy(
                emb_hbm.at[idx_v.at[j]],
                rows_v.at[pl.ds(j * _CHUNK, _CHUNK)],
                sem,
            )
        )
    for c in copies:
        c.wait()
    pltpu.sync_copy(rows_v, out_hbm.at[pl.ds(wid * _TPT, _TPT)])


def kernel(word_idx, table, ngram_idx, ngram_cnt):
    idxp = jnp.pad(ngram_idx, ((0, 0), (0, _GPAD - ngram_idx.shape[1])))
    idxp = idxp.reshape(_NW, _WPT * _GPAD)
    cntb = jnp.broadcast_to(ngram_cnt[:, None], (_V, _E))
    tok_idx = word_idx.reshape(_NW, _NCHUNK, _CHUNK)
    emb = _build_word_emb(table, idxp, cntb)
    out = _expand(emb, tok_idx)
    return out.reshape(word_idx.shape + (_E,))


# speedup vs baseline: 9.6540x; 1.0396x over previous
import functools

import jax
import jax.numpy as jnp
from jax import lax
from jax.experimental import pallas as pl
from jax.experimental.pallas import tpu as pltpu
from jax.experimental.pallas import tpu_sc as plsc

_info = plsc.get_sparse_core_info()
_NC, _NS, _L = _info.num_cores, _info.num_subcores, _info.num_lanes
_NW = _NC * _NS

_V = 64
_E = 64
_GPAD = 8
_WPS = _V // _NS
_TOK = 1024 * 20
_TPT = _TOK // _NW
_CHUNK = 128
_NCHUNK = _TPT // _CHUNK

_mesh = plsc.VectorSubcoreMesh(core_axis_name="c", subcore_axis_name="s")
_params = pltpu.CompilerParams(use_tc_tiling_on_sc=False)


@functools.partial(
    pl.kernel,
    mesh=_mesh,
    compiler_params=_params,
    out_type=(
        jax.ShapeDtypeStruct((_NC, _V, _E), jnp.float32),
        jax.ShapeDtypeStruct((_TOK, _E), jnp.float32),
    ),
    scratch_types=[
        pltpu.VMEM((_WPS * _GPAD,), jnp.int32),
        pltpu.VMEM((_WPS * _GPAD, _E), jnp.float32),
        pltpu.VMEM((_WPS, _E), jnp.float32),
        pltpu.VMEM((_WPS, _E), jnp.float32),
        pltpu.VMEM((_NCHUNK, _CHUNK), jnp.int32),
        pltpu.VMEM((_TPT, _E), jnp.float32),
        pltpu.SemaphoreType.DMA,
        pltpu.SemaphoreType.DMA,
    ],
)
def _ngram_embed(table_hbm, idxp_hbm, cntb_hbm, tok_hbm, emb_hbm, out_hbm,
                 idxA_v, rowsA_v, cnt_v, emb_v, idx_v, rows_v, semA, semB):
    cid = lax.axis_index("c")
    sid = lax.axis_index("s")
    wid = sid * _NC + cid
    cp_idx = pltpu.async_copy(tok_hbm.at[wid], idx_v, semB)
    pltpu.sync_copy(idxp_hbm.at[sid], idxA_v)
    pltpu.async_copy(table_hbm.at[idxA_v], rowsA_v, semA).wait()
    pltpu.sync_copy(cntb_hbm.at[pl.ds(sid * _WPS, _WPS)], cnt_v)
    for wloc in range(_WPS):
        for c in range(_E // _L):
            acc = rowsA_v[_GPAD * wloc, pl.ds(c * _L, _L)]
            for l in range(1, _GPAD):
                acc = acc + rowsA_v[_GPAD * wloc + l, pl.ds(c * _L, _L)]
            emb_v[wloc, pl.ds(c * _L, _L)] = acc / cnt_v[wloc, pl.ds(c * _L, _L)]
    pltpu.sync_copy(emb_v, emb_hbm.at[cid].at[pl.ds(sid * _WPS, _WPS)])
    plsc.subcore_barrier()
    cp_idx.wait()
    copies = []
    for j in range(_NCHUNK):
        copies.append(
            pltpu.async_copy(
                emb_hbm.at[cid].at[idx_v.at[j]],
                rows_v.at[pl.ds(j * _CHUNK, _CHUNK)],
                semA,
            )
        )
    for c in copies:
        c.wait()
    pltpu.sync_copy(rows_v, out_hbm.at[pl.ds(wid * _TPT, _TPT)])


def kernel(word_idx, table, ngram_idx, ngram_cnt):
    idxp = jnp.pad(ngram_idx, ((0, 0), (0, _GPAD - ngram_idx.shape[1])))
    idxp = idxp.reshape(_NS, _WPS * _GPAD)
    cntb = jnp.broadcast_to(ngram_cnt[:, None], (_V, _E))
    tok_idx = word_idx.reshape(_NW, _NCHUNK, _CHUNK)
    _, out = _ngram_embed(table, idxp, cntb, tok_idx)
    return out.reshape(word_idx.shape + (_E,))
